# baseline (speedup 1.0000x reference)
import functools

import jax
import jax.numpy as jnp
from jax import lax
from jax.experimental import pallas as pl
from jax.experimental.pallas import tpu as pltpu
from jax.experimental.pallas import tpu_sc as plsc

D_MODEL = 64
B_TOKENS = 4096 * 200


def _make_gather(vocab: int, d: int, b: int):
  info = plsc.get_sparse_core_info()
  nw = info.num_cores * info.num_subcores
  b_per_w = b // nw
  chunk = 128
  nchunk = b_per_w // chunk
  mesh = plsc.VectorSubcoreMesh(core_axis_name="c", subcore_axis_name="s")

  @functools.partial(
      pl.kernel,
      mesh=mesh,
      out_type=jax.ShapeDtypeStruct((b, d), jnp.float32),
      scratch_types=[
          pltpu.VMEM((b_per_w,), jnp.int32),
          pltpu.VMEM((chunk, d), jnp.float32),
          pltpu.SemaphoreType.DMA,
      ],
      compiler_params=pltpu.CompilerParams(use_tc_tiling_on_sc=False),
  )
  def gather(table_hbm, idx_hbm, out_hbm, idx_v, rows_v, sem):
    wid = lax.axis_index("s") * info.num_cores + lax.axis_index("c")
    base = wid * b_per_w
    pltpu.sync_copy(idx_hbm.at[pl.ds(base, b_per_w)], idx_v)

    def body(i, carry):
      off = i * chunk
      pltpu.async_copy(
          table_hbm.at[idx_v.at[pl.ds(off, chunk)]], rows_v, sem
      ).wait()
      pltpu.sync_copy(rows_v, out_hbm.at[pl.ds(base + off, chunk)])
      return carry

    lax.fori_loop(0, nchunk, body, 0)

  return gather


def kernel(x, table):
  b, t = x.shape
  vocab, d = table.shape
  idx = x.reshape(-1).astype(jnp.int32)
  out = _make_gather(vocab, d, b * t)(table, idx)
  return out.reshape(b, t, d)


# speedup vs baseline: 1.1168x; 1.1168x over previous
import functools

import jax
import jax.numpy as jnp
from jax import lax
from jax.experimental import pallas as pl
from jax.experimental.pallas import tpu as pltpu
from jax.experimental.pallas import tpu_sc as plsc

CHUNK = 128
NB = 8
A = 4


def _make_gather(vocab: int, d: int, b: int):
  info = plsc.get_sparse_core_info()
  nw = info.num_cores * info.num_subcores
  b_per_w = b // nw
  nchunk = b_per_w // CHUNK
  ngroup = nchunk // NB
  assert nchunk % NB == 0 and A < NB
  mesh = plsc.VectorSubcoreMesh(core_axis_name="c", subcore_axis_name="s")

  @functools.partial(
      pl.kernel,
      mesh=mesh,
      out_type=jax.ShapeDtypeStruct((b, d), jnp.float32),
      scratch_types=(
          [pltpu.VMEM((b_per_w,), jnp.int32),
           pltpu.VMEM((NB * CHUNK, d), jnp.float32)]
          + [pltpu.SemaphoreType.DMA] * (2 * NB)
      ),
      compiler_params=pltpu.CompilerParams(use_tc_tiling_on_sc=False),
  )
  def gather(table_hbm, idx_hbm, out_hbm, idx_v, bufs, *sems):
    gsem, osem = sems[:NB], sems[NB:]
    wid = lax.axis_index("s") * info.num_cores + lax.axis_index("c")
    base = wid * b_per_w
    pltpu.sync_copy(idx_hbm.at[pl.ds(base, b_per_w)], idx_v)

    def buf(s):
      return bufs.at[pl.ds(s * CHUNK, CHUNK)]

    def issue_gather(p, s):
      pltpu.async_copy(
          table_hbm.at[idx_v.at[pl.ds(p * CHUNK, CHUNK)]], buf(s), gsem[s])

    def wait_gather(p, s):
      pltpu.make_async_copy(
          table_hbm.at[idx_v.at[pl.ds(p * CHUNK, CHUNK)]], buf(s),
          gsem[s]).wait()

    def issue_out(p, s):
      pltpu.async_copy(
          buf(s), out_hbm.at[pl.ds(base + p * CHUNK, CHUNK)], osem[s])

    def wait_out(p, s):
      pltpu.make_async_copy(
          buf(s), out_hbm.at[pl.ds(base + p * CHUNK, CHUNK)],
          osem[s]).wait()

    for s in range(A):
      issue_gather(s, s)

    for s in range(NB):
      wait_gather(s, s)
      issue_out(s, s)
      p = s + A
      if p < NB:
        issue_gather(p, p)
      else:
        wait_out(p - NB, p - NB)
        issue_gather(p, p - NB)

    def body(g, carry):
      i0 = g * NB
      for s in range(NB):
        i = i0 + s
        wait_gather(i, s)
        issue_out(i, s)
        sp = (s + A) % NB
        wait_out(i + A - NB, sp)
        issue_gather(i + A, sp)
      return carry

    lax.fori_loop(1, ngroup - 1, body, 0)

    i0 = (ngroup - 1) * NB
    for s in range(NB):
      i = i0 + s
      wait_gather(i, s)
      issue_out(i, s)
      p = i + A
      if p < nchunk:
        sp = (s + A) % NB
        wait_out(p - NB, sp)
        issue_gather(p, sp)

    for s in range(NB):
      wait_out(i0 + s, s)

  return gather


def kernel(x, table):
  b, t = x.shape
  vocab, d = table.shape
  idx = x.reshape(-1).astype(jnp.int32)
  out = _make_gather(vocab, d, b * t)(table, idx)
  return out.reshape(b, t, d)
